# Initial kernel scaffold; baseline (speedup 1.0000x reference)
import functools

import jax
import jax.numpy as jnp
from jax import lax
from jax.experimental import pallas as pl
from jax.experimental.pallas import tpu as pltpu
from jax.experimental.pallas import tpu_sc as plsc

N = 10000
E = 320000
WIDTH = 128
NLAYERS = 4
VOCAB = 128
MAX_POS = 256
EPS = 1e-5

NC = 2
NS = 16
NW = NC * NS
EPC = E // NC
EPW = E // NW
CH = 80
NCHUNK = EPW // CH
RPW = N // NS
ZR = 125
DEGW = 16

_sc_mesh = plsc.VectorSubcoreMesh(core_axis_name="c", subcore_axis_name="s")


@functools.partial(
    pl.kernel,
    out_type=jax.ShapeDtypeStruct((NC * N, DEGW), jnp.float32),
    mesh=_sc_mesh,
    scratch_types=[
        pltpu.VMEM((CH,), jnp.int32),
        pltpu.VMEM((CH, DEGW), jnp.float32),
        pltpu.VMEM((RPW, DEGW), jnp.float32),
        pltpu.VMEM_SHARED((N, DEGW), jnp.float32),
    ],
)
def _sc_deg(dst_hbm, out_hbm, didx, ones, zbuf, acc):
    c = lax.axis_index("c")
    s = lax.axis_index("s")

    def fill_ones(i, carry):
        ones[i, pl.ds(0, 16)] = jnp.full((16,), 1.0, jnp.float32)
        return carry

    lax.fori_loop(0, CH, fill_ones, 0)

    def fill_zero(i, carry):
        zbuf[i, pl.ds(0, 16)] = jnp.zeros((16,), jnp.float32)
        return carry

    lax.fori_loop(0, RPW, fill_zero, 0)
    pltpu.sync_copy(zbuf, acc.at[pl.ds(s * RPW, RPW)])
    plsc.subcore_barrier()

    base = c * EPC + s * EPW

    def body(k, carry):
        pltpu.sync_copy(dst_hbm.at[pl.ds(base + k * CH, CH)], didx)
        pltpu.sync_copy(ones, acc.at[didx], add=True)
        return carry

    lax.fori_loop(0, NCHUNK, body, 0)
    plsc.subcore_barrier()
    pltpu.sync_copy(acc.at[pl.ds(s * RPW, RPW)],
                    out_hbm.at[pl.ds(c * N + s * RPW, RPW)])


@functools.partial(
    pl.kernel,
    out_type=jax.ShapeDtypeStruct((NC * N, WIDTH), jnp.float32),
    mesh=_sc_mesh,
    scratch_types=[
        pltpu.VMEM((CH,), jnp.int32),
        pltpu.VMEM((CH,), jnp.int32),
        pltpu.VMEM((CH, WIDTH), jnp.float32),
        pltpu.VMEM((ZR, WIDTH), jnp.float32),
        pltpu.VMEM_SHARED((N, WIDTH), jnp.float32),
        pltpu.SemaphoreType.DMA,
    ],
)
def _sc_agg(h_hbm, src_hbm, dst_hbm, out_hbm, sidx, didx, rows, zbuf, acc, sem):
    c = lax.axis_index("c")
    s = lax.axis_index("s")

    def fill_zero(i, carry):
        def col(j, carry2):
            zbuf[i, pl.ds(j * 16, 16)] = jnp.zeros((16,), jnp.float32)
            return carry2
        return lax.fori_loop(0, WIDTH // 16, col, carry)

    lax.fori_loop(0, ZR, fill_zero, 0)
    for j in range(RPW // ZR):
        pltpu.sync_copy(zbuf, acc.at[pl.ds(s * RPW + j * ZR, ZR)])
    plsc.subcore_barrier()

    base = c * EPC + s * EPW

    def body(k, carry):
        off = base + k * CH
        pltpu.sync_copy(src_hbm.at[pl.ds(off, CH)], sidx)
        pltpu.sync_copy(dst_hbm.at[pl.ds(off, CH)], didx)
        pltpu.async_copy(h_hbm.at[sidx], rows, sem).wait()
        pltpu.sync_copy(rows, acc.at[didx], add=True)
        return carry

    lax.fori_loop(0, NCHUNK, body, 0)
    plsc.subcore_barrier()
    pltpu.sync_copy(acc.at[pl.ds(s * RPW, RPW)],
                    out_hbm.at[pl.ds(c * N + s * RPW, RPW)])


RB = 2000


def _tc_embed_body(x_ref, p_ref, vt_ref, pt_ref, g_ref, b_ref, m_ref, v_ref,
                   out_ref):
    xv = x_ref[...]
    pv = p_ref[...]
    oh_x = (xv == lax.broadcasted_iota(jnp.int32, (1, VOCAB), 1)
            ).astype(jnp.float32)
    oh_p = (pv == lax.broadcasted_iota(jnp.int32, (1, MAX_POS), 1)
            ).astype(jnp.float32)
    h = (jnp.dot(oh_x, vt_ref[...], preferred_element_type=jnp.float32,
                 precision=lax.Precision.HIGHEST)
         + jnp.dot(oh_p, pt_ref[...], preferred_element_type=jnp.float32,
                   precision=lax.Precision.HIGHEST))
    scale = g_ref[...] * lax.rsqrt(v_ref[...] + EPS)
    out_ref[...] = (h - m_ref[...]) * scale + b_ref[...]


_row_spec = pl.BlockSpec((RB, 1), lambda i: (i, 0))
_full_spec = lambda r, c: pl.BlockSpec((r, c), lambda i: (0, 0))
_h_spec = pl.BlockSpec((RB, WIDTH), lambda i: (i, 0))

_tc_embed = pl.pallas_call(
    _tc_embed_body,
    grid=(N // RB,),
    in_specs=[
        _row_spec, _row_spec,
        _full_spec(VOCAB, WIDTH), _full_spec(MAX_POS, WIDTH),
        _full_spec(1, WIDTH), _full_spec(1, WIDTH),
        _full_spec(1, WIDTH), _full_spec(1, WIDTH),
    ],
    out_specs=_h_spec,
    out_shape=jax.ShapeDtypeStruct((N, WIDTH), jnp.float32),
)


def _layer_math(agg_ref, deg_ref, hb_ref, wr_ref, br_ref, wt_ref):
    deg = jnp.maximum(deg_ref[0, :, 0:1] + deg_ref[1, :, 0:1], 1.0)
    t = (agg_ref[0] + agg_ref[1]) / deg
    h = (jnp.dot(t, wr_ref[...], preferred_element_type=jnp.float32,
                 precision=lax.Precision.HIGHEST)
         + br_ref[...]
         + jnp.dot(hb_ref[...], wt_ref[...], preferred_element_type=jnp.float32,
                   precision=lax.Precision.HIGHEST))
    return jnp.maximum(h, 0.0)


def _tc_layer_body(agg_ref, deg_ref, hb_ref, wr_ref, br_ref, wt_ref,
                   g_ref, b_ref, m_ref, v_ref, h_ref, hbn_ref):
    h = _layer_math(agg_ref, deg_ref, hb_ref, wr_ref, br_ref, wt_ref)
    h_ref[...] = h
    scale = g_ref[...] * lax.rsqrt(v_ref[...] + EPS)
    hbn_ref[...] = (h - m_ref[...]) * scale + b_ref[...]


def _tc_last_body(agg_ref, deg_ref, hb_ref, wr_ref, br_ref, wt_ref, h_ref):
    h_ref[...] = _layer_math(agg_ref, deg_ref, hb_ref, wr_ref, br_ref, wt_ref)


_agg_spec = pl.BlockSpec((NC, RB, WIDTH), lambda i: (0, i, 0))
_deg_spec = pl.BlockSpec((NC, RB, DEGW), lambda i: (0, i, 0))
_w_spec = _full_spec(WIDTH, WIDTH)
_vec_spec = _full_spec(1, WIDTH)

_tc_layer = pl.pallas_call(
    _tc_layer_body,
    grid=(N // RB,),
    in_specs=[_agg_spec, _deg_spec, _h_spec, _w_spec, _vec_spec, _w_spec,
              _vec_spec, _vec_spec, _vec_spec, _vec_spec],
    out_specs=[_h_spec, _h_spec],
    out_shape=[jax.ShapeDtypeStruct((N, WIDTH), jnp.float32),
               jax.ShapeDtypeStruct((N, WIDTH), jnp.float32)],
)

_tc_last = pl.pallas_call(
    _tc_last_body,
    grid=(N // RB,),
    in_specs=[_agg_spec, _deg_spec, _h_spec, _w_spec, _vec_spec, _w_spec],
    out_specs=_h_spec,
    out_shape=jax.ShapeDtypeStruct((N, WIDTH), jnp.float32),
)


def kernel(x, pos, edge_index, edge_attr, vocab_table, pos_table,
           bn_gamma, bn_beta, bn_mean, bn_var, W_rel, b_rel, W_root):
    del edge_attr
    src = edge_index[0].astype(jnp.int32)
    dst = edge_index[1].astype(jnp.int32)
    x2 = x.astype(jnp.int32).reshape(N, 1)
    p2 = pos.astype(jnp.int32).reshape(N, 1)

    degp = _sc_deg(dst).reshape(NC, N, DEGW)

    row = lambda a: a.reshape(1, WIDTH)
    hb = _tc_embed(x2, p2, vocab_table, pos_table,
                   row(bn_gamma[0]), row(bn_beta[0]),
                   row(bn_mean[0]), row(bn_var[0]))

    h = None
    for i in range(NLAYERS):
        aggp = _sc_agg(hb, src, dst).reshape(NC, N, WIDTH)
        wr = W_rel[i].T
        wt = W_root[i].T
        br = b_rel[i].reshape(1, WIDTH)
        if i + 1 < NLAYERS:
            h, hb = _tc_layer(aggp, degp, hb, wr, br, wt,
                              row(bn_gamma[i + 1]), row(bn_beta[i + 1]),
                              row(bn_mean[i + 1]), row(bn_var[i + 1]))
        else:
            h = _tc_last(aggp, degp, hb, wr, br, wt)
    return h


# speedup vs baseline: 1.0539x; 1.0539x over previous
import functools

import jax
import jax.numpy as jnp
from jax import lax
from jax.experimental import pallas as pl
from jax.experimental.pallas import tpu as pltpu
from jax.experimental.pallas import tpu_sc as plsc

N = 10000
E = 320000
WIDTH = 128
NLAYERS = 4
VOCAB = 128
MAX_POS = 256
EPS = 1e-5

NC = 2
NS = 16
NW = NC * NS
EPC = E // NC
EPW = E // NW
CH = 80
NCHUNK = EPW // CH
NPAD = 10240
RPW = NPAD // NS
ZR = 128
DEGW = 16

def _sc_deg_body(dst_hbm, out_hbm, didx, ones, zbuf, acc):
    c = lax.axis_index("c")
    s = lax.axis_index("s")

    def fill_ones(i, carry):
        ones[i, pl.ds(0, 16)] = jnp.full((16,), 1.0, jnp.float32)
        return carry

    lax.fori_loop(0, CH, fill_ones, 0)

    def fill_zero(i, carry):
        zbuf[i, pl.ds(0, 16)] = jnp.zeros((16,), jnp.float32)
        return carry

    lax.fori_loop(0, RPW, fill_zero, 0)
    pltpu.sync_copy(zbuf, acc.at[pl.ds(s * RPW, RPW)])
    plsc.subcore_barrier()

    base = c * EPC + s * EPW

    def body(k, carry):
        pltpu.sync_copy(dst_hbm.at[pl.ds(base + k * CH, CH)], didx)
        pltpu.sync_copy(ones, acc.at[didx], add=True)
        return carry

    lax.fori_loop(0, NCHUNK, body, 0)
    plsc.subcore_barrier()
    pltpu.sync_copy(acc.at[pl.ds(s * RPW, RPW)], zbuf)
    pltpu.sync_copy(zbuf, out_hbm.at[pl.ds(c * NPAD + s * RPW, RPW)])


def _sc_agg_body(h_hbm, src_hbm, dst_hbm, out_hbm, sidx, didx, rows, zbuf,
                 acc, sem):
    c = lax.axis_index("c")
    s = lax.axis_index("s")

    def fill_zero(i, carry):
        def col(j, carry2):
            zbuf[i, pl.ds(j * 16, 16)] = jnp.zeros((16,), jnp.float32)
            return carry2
        return lax.fori_loop(0, WIDTH // 16, col, carry)

    lax.fori_loop(0, ZR, fill_zero, 0)
    for j in range(RPW // ZR):
        pltpu.sync_copy(zbuf, acc.at[pl.ds(s * RPW + j * ZR, ZR)])
    plsc.subcore_barrier()

    base = c * EPC + s * EPW

    def body(k, carry):
        off = base + k * CH
        pltpu.sync_copy(src_hbm.at[pl.ds(off, CH)], sidx)
        pltpu.sync_copy(dst_hbm.at[pl.ds(off, CH)], didx)
        pltpu.async_copy(h_hbm.at[sidx], rows, sem).wait()
        pltpu.sync_copy(rows, acc.at[didx], add=True)
        return carry

    lax.fori_loop(0, NCHUNK, body, 0)
    plsc.subcore_barrier()
    for j in range(RPW // ZR):
        pltpu.sync_copy(acc.at[pl.ds(s * RPW + j * ZR, ZR)], zbuf)
        pltpu.sync_copy(zbuf, out_hbm.at[pl.ds(c * NPAD + s * RPW + j * ZR, ZR)])


@functools.cache
def _sc_kernels():
    mesh = plsc.VectorSubcoreMesh(core_axis_name="c", subcore_axis_name="s",
                                  num_cores=NC, num_subcores=NS)
    sc_deg = pl.kernel(
        _sc_deg_body,
        out_type=jax.ShapeDtypeStruct((NC * NPAD, DEGW), jnp.float32),
        mesh=mesh,
        scratch_types=[
            pltpu.VMEM((CH,), jnp.int32),
            pltpu.VMEM((CH, DEGW), jnp.float32),
            pltpu.VMEM((RPW, DEGW), jnp.float32),
            pltpu.VMEM_SHARED((NPAD, DEGW), jnp.float32),
        ],
    )
    sc_agg = pl.kernel(
        _sc_agg_body,
        out_type=jax.ShapeDtypeStruct((NC * NPAD, WIDTH), jnp.float32),
        mesh=mesh,
        scratch_types=[
            pltpu.VMEM((CH,), jnp.int32),
            pltpu.VMEM((CH,), jnp.int32),
            pltpu.VMEM((CH, WIDTH), jnp.float32),
            pltpu.VMEM((ZR, WIDTH), jnp.float32),
            pltpu.VMEM_SHARED((NPAD, WIDTH), jnp.float32),
            pltpu.SemaphoreType.DMA,
        ],
    )
    return sc_deg, sc_agg


RB = 1024


def _tc_embed_body(x_ref, p_ref, vt_ref, pt_ref, g_ref, b_ref, m_ref, v_ref,
                   out_ref):
    xv = x_ref[...]
    pv = p_ref[...]
    oh_x = (xv == lax.broadcasted_iota(jnp.int32, (1, VOCAB), 1)
            ).astype(jnp.float32)
    oh_p = (pv == lax.broadcasted_iota(jnp.int32, (1, MAX_POS), 1)
            ).astype(jnp.float32)
    h = (jnp.dot(oh_x, vt_ref[...], preferred_element_type=jnp.float32,
                 precision=lax.Precision.HIGHEST)
         + jnp.dot(oh_p, pt_ref[...], preferred_element_type=jnp.float32,
                   precision=lax.Precision.HIGHEST))
    scale = g_ref[...] * lax.rsqrt(v_ref[...] + EPS)
    out_ref[...] = (h - m_ref[...]) * scale + b_ref[...]


_row_spec = pl.BlockSpec((RB, 1), lambda i: (i, 0))
_full_spec = lambda r, c: pl.BlockSpec((r, c), lambda i: (0, 0))
_h_spec = pl.BlockSpec((RB, WIDTH), lambda i: (i, 0))

_tc_embed = pl.pallas_call(
    _tc_embed_body,
    grid=(NPAD // RB,),
    in_specs=[
        _row_spec, _row_spec,
        _full_spec(VOCAB, WIDTH), _full_spec(MAX_POS, WIDTH),
        _full_spec(1, WIDTH), _full_spec(1, WIDTH),
        _full_spec(1, WIDTH), _full_spec(1, WIDTH),
    ],
    out_specs=_h_spec,
    out_shape=jax.ShapeDtypeStruct((NPAD, WIDTH), jnp.float32),
)


def _layer_math(agg_ref, deg_ref, hb_ref, wr_ref, br_ref, wt_ref):
    deg = jnp.maximum(deg_ref[0, :, 0:1] + deg_ref[1, :, 0:1], 1.0)
    t = (agg_ref[0] + agg_ref[1]) / deg
    h = (jnp.dot(t, wr_ref[...], preferred_element_type=jnp.float32,
                 precision=lax.Precision.HIGHEST)
         + br_ref[...]
         + jnp.dot(hb_ref[...], wt_ref[...], preferred_element_type=jnp.float32,
                   precision=lax.Precision.HIGHEST))
    return jnp.maximum(h, 0.0)


def _tc_layer_body(agg_ref, deg_ref, hb_ref, wr_ref, br_ref, wt_ref,
                   g_ref, b_ref, m_ref, v_ref, h_ref, hbn_ref):
    h = _layer_math(agg_ref, deg_ref, hb_ref, wr_ref, br_ref, wt_ref)
    h_ref[...] = h
    scale = g_ref[...] * lax.rsqrt(v_ref[...] + EPS)
    hbn_ref[...] = (h - m_ref[...]) * scale + b_ref[...]


def _tc_last_body(agg_ref, deg_ref, hb_ref, wr_ref, br_ref, wt_ref, h_ref):
    h_ref[...] = _layer_math(agg_ref, deg_ref, hb_ref, wr_ref, br_ref, wt_ref)


_agg_spec = pl.BlockSpec((NC, RB, WIDTH), lambda i: (0, i, 0))
_deg_spec = pl.BlockSpec((NC, RB, DEGW), lambda i: (0, i, 0))
_w_spec = _full_spec(WIDTH, WIDTH)
_vec_spec = _full_spec(1, WIDTH)

_tc_layer = pl.pallas_call(
    _tc_layer_body,
    grid=(NPAD // RB,),
    in_specs=[_agg_spec, _deg_spec, _h_spec, _w_spec, _vec_spec, _w_spec,
              _vec_spec, _vec_spec, _vec_spec, _vec_spec],
    out_specs=[_h_spec, _h_spec],
    out_shape=[jax.ShapeDtypeStruct((NPAD, WIDTH), jnp.float32),
               jax.ShapeDtypeStruct((NPAD, WIDTH), jnp.float32)],
)

_tc_last = pl.pallas_call(
    _tc_last_body,
    grid=(NPAD // RB,),
    in_specs=[_agg_spec, _deg_spec, _h_spec, _w_spec, _vec_spec, _w_spec],
    out_specs=_h_spec,
    out_shape=jax.ShapeDtypeStruct((NPAD, WIDTH), jnp.float32),
)


def kernel(x, pos, edge_index, edge_attr, vocab_table, pos_table,
           bn_gamma, bn_beta, bn_mean, bn_var, W_rel, b_rel, W_root):
    del edge_attr
    src = edge_index[0].astype(jnp.int32)
    dst = edge_index[1].astype(jnp.int32)
    padn = jnp.zeros((NPAD - N,), jnp.int32)
    x2 = jnp.concatenate([x.astype(jnp.int32), padn]).reshape(NPAD, 1)
    p2 = jnp.concatenate([pos.astype(jnp.int32), padn]).reshape(NPAD, 1)

    deg_tmp = jax.ops.segment_sum(jnp.ones((E,), jnp.float32), dst,
                                  num_segments=NPAD)
    deg_tmp = jnp.broadcast_to(deg_tmp[:, None], (NPAD, DEGW))
    degp = jnp.stack([deg_tmp, jnp.zeros_like(deg_tmp)])

    row = lambda a: a.reshape(1, WIDTH)
    hb = _tc_embed(x2, p2, vocab_table, pos_table,
                   row(bn_gamma[0]), row(bn_beta[0]),
                   row(bn_mean[0]), row(bn_var[0]))

    h = None
    for i in range(NLAYERS):
        agg_tmp = jax.ops.segment_sum(hb[src], dst, num_segments=NPAD)
        aggp = jnp.stack([agg_tmp, jnp.zeros_like(agg_tmp)])
        wr = W_rel[i].T
        wt = W_root[i].T
        br = b_rel[i].reshape(1, WIDTH)
        if i + 1 < NLAYERS:
            h, hb = _tc_layer(aggp, degp, hb, wr, br, wt,
                              row(bn_gamma[i + 1]), row(bn_beta[i + 1]),
                              row(bn_mean[i + 1]), row(bn_var[i + 1]))
        else:
            h = _tc_last(aggp, degp, hb, wr, br, wt)
    return h[:N]


# speedup vs baseline: 1.0540x; 1.0001x over previous
import functools

import jax
import jax.numpy as jnp
from jax import lax
from jax.experimental import pallas as pl
from jax.experimental.pallas import tpu as pltpu
from jax.experimental.pallas import tpu_sc as plsc

N = 10000
E = 320000
WIDTH = 128
NLAYERS = 4
VOCAB = 128
MAX_POS = 256
EPS = 1e-5

NC = 2
NS = 16
NW = NC * NS
EPC = E // NC
EPW = E // NW
CH = 80
NCHUNK = EPW // CH
NPAD = 10240
RPW = NPAD // NS
ZR = 128
DEGW = 16

def _sc_deg_body(dst_hbm, out_hbm, didx, ones, zbuf, bidx, acc):
    c = lax.axis_index("c")
    s = lax.axis_index("s")

    def fill_ones(i, carry):
        ones[i, pl.ds(0, 16)] = jnp.full((16,), 1.0, jnp.float32)
        return carry

    lax.fori_loop(0, CH, fill_ones, 0)

    def fill_zero(i, carry):
        zbuf[i, pl.ds(0, 16)] = jnp.zeros((16,), jnp.float32)
        return carry

    lax.fori_loop(0, ZR, fill_zero, 0)

    def set_bidx(j):
        base = s * RPW + j * ZR
        for t in range(ZR // 16):
            bidx[pl.ds(t * 16, 16)] = base + t * 16 + lax.iota(jnp.int32, 16)

    for j in range(RPW // ZR):
        set_bidx(j)
        pltpu.sync_copy(zbuf, acc.at[bidx])
    plsc.subcore_barrier()

    base_e = c * EPC + s * EPW

    def body(k, carry):
        pltpu.sync_copy(dst_hbm.at[pl.ds(base_e + k * CH, CH)], didx)
        for g in range(CH // 16):
            idx16 = didx[pl.ds(g * 16, 16)]
            pltpu.sync_copy(ones.at[pl.ds(g * 16, 16)], acc.at[idx16],
                            add=True)
        return carry

    lax.fori_loop(0, NCHUNK, body, 0)
    plsc.subcore_barrier()

    for j in range(RPW // ZR):
        set_bidx(j)
        pltpu.sync_copy(acc.at[bidx], zbuf)
        pltpu.sync_copy(zbuf,
                        out_hbm.at[pl.ds(c * NPAD + s * RPW + j * ZR, ZR)])


def _sc_agg_body(h_hbm, src_hbm, dst_hbm, out_hbm, sidx, didx, rows, zbuf,
                 acc, sem):
    c = lax.axis_index("c")
    s = lax.axis_index("s")

    def fill_zero(i, carry):
        def col(j, carry2):
            zbuf[i, pl.ds(j * 16, 16)] = jnp.zeros((16,), jnp.float32)
            return carry2
        return lax.fori_loop(0, WIDTH // 16, col, carry)

    lax.fori_loop(0, ZR, fill_zero, 0)
    for j in range(RPW // ZR):
        pltpu.sync_copy(zbuf, acc.at[pl.ds(s * RPW + j * ZR, ZR)])
    plsc.subcore_barrier()

    base = c * EPC + s * EPW

    def body(k, carry):
        off = base + k * CH
        pltpu.sync_copy(src_hbm.at[pl.ds(off, CH)], sidx)
        pltpu.sync_copy(dst_hbm.at[pl.ds(off, CH)], didx)
        pltpu.async_copy(h_hbm.at[sidx], rows, sem).wait()
        for g in range(CH // 16):
            idx16 = didx[pl.ds(g * 16, 16)]
            pltpu.sync_copy(rows.at[pl.ds(g * 16, 16)], acc.at[idx16],
                            add=True)
        return carry

    lax.fori_loop(0, NCHUNK, body, 0)
    plsc.subcore_barrier()
    for j in range(RPW // ZR):
        pltpu.sync_copy(acc.at[pl.ds(s * RPW + j * ZR, ZR)], zbuf)
        pltpu.sync_copy(zbuf, out_hbm.at[pl.ds(c * NPAD + s * RPW + j * ZR, ZR)])


@functools.cache
def _sc_kernels():
    mesh = plsc.VectorSubcoreMesh(core_axis_name="c", subcore_axis_name="s",
                                  num_cores=NC, num_subcores=NS)
    sc_deg = pl.kernel(
        _sc_deg_body,
        out_type=jax.ShapeDtypeStruct((NC * NPAD, DEGW), jnp.float32),
        mesh=mesh,
        scratch_types=[
            pltpu.VMEM((CH,), jnp.int32),
            pltpu.VMEM((CH, DEGW), jnp.float32),
            pltpu.VMEM((ZR, DEGW), jnp.float32),
            pltpu.VMEM((ZR,), jnp.int32),
            pltpu.VMEM_SHARED((NPAD, DEGW), jnp.float32),
        ],
    )
    sc_agg = pl.kernel(
        _sc_agg_body,
        out_type=jax.ShapeDtypeStruct((NC * NPAD, WIDTH), jnp.float32),
        mesh=mesh,
        scratch_types=[
            pltpu.VMEM((CH,), jnp.int32),
            pltpu.VMEM((CH,), jnp.int32),
            pltpu.VMEM((CH, WIDTH), jnp.float32),
            pltpu.VMEM((ZR, WIDTH), jnp.float32),
            pltpu.VMEM_SHARED((NPAD, WIDTH), jnp.float32),
            pltpu.SemaphoreType.DMA,
        ],
    )
    return sc_deg, sc_agg


RB = 1024


def _tc_embed_body(x_ref, p_ref, vt_ref, pt_ref, g_ref, b_ref, m_ref, v_ref,
                   out_ref):
    xv = x_ref[...]
    pv = p_ref[...]
    oh_x = (xv == lax.broadcasted_iota(jnp.int32, (1, VOCAB), 1)
            ).astype(jnp.float32)
    oh_p = (pv == lax.broadcasted_iota(jnp.int32, (1, MAX_POS), 1)
            ).astype(jnp.float32)
    h = (jnp.dot(oh_x, vt_ref[...], preferred_element_type=jnp.float32,
                 precision=lax.Precision.HIGHEST)
         + jnp.dot(oh_p, pt_ref[...], preferred_element_type=jnp.float32,
                   precision=lax.Precision.HIGHEST))
    scale = g_ref[...] * lax.rsqrt(v_ref[...] + EPS)
    out_ref[...] = (h - m_ref[...]) * scale + b_ref[...]


_row_spec = pl.BlockSpec((RB, 1), lambda i: (i, 0))
_full_spec = lambda r, c: pl.BlockSpec((r, c), lambda i: (0, 0))
_h_spec = pl.BlockSpec((RB, WIDTH), lambda i: (i, 0))

_tc_embed = pl.pallas_call(
    _tc_embed_body,
    grid=(NPAD // RB,),
    in_specs=[
        _row_spec, _row_spec,
        _full_spec(VOCAB, WIDTH), _full_spec(MAX_POS, WIDTH),
        _full_spec(1, WIDTH), _full_spec(1, WIDTH),
        _full_spec(1, WIDTH), _full_spec(1, WIDTH),
    ],
    out_specs=_h_spec,
    out_shape=jax.ShapeDtypeStruct((NPAD, WIDTH), jnp.float32),
)


def _layer_math(agg_ref, deg_ref, hb_ref, wr_ref, br_ref, wt_ref):
    deg = jnp.maximum(deg_ref[0, :, 0:1] + deg_ref[1, :, 0:1], 1.0)
    t = (agg_ref[0] + agg_ref[1]) / deg
    h = (jnp.dot(t, wr_ref[...], preferred_element_type=jnp.float32,
                 precision=lax.Precision.HIGHEST)
         + br_ref[...]
         + jnp.dot(hb_ref[...], wt_ref[...], preferred_element_type=jnp.float32,
                   precision=lax.Precision.HIGHEST))
    return jnp.maximum(h, 0.0)


def _tc_layer_body(agg_ref, deg_ref, hb_ref, wr_ref, br_ref, wt_ref,
                   g_ref, b_ref, m_ref, v_ref, h_ref, hbn_ref):
    h = _layer_math(agg_ref, deg_ref, hb_ref, wr_ref, br_ref, wt_ref)
    h_ref[...] = h
    scale = g_ref[...] * lax.rsqrt(v_ref[...] + EPS)
    hbn_ref[...] = (h - m_ref[...]) * scale + b_ref[...]


def _tc_last_body(agg_ref, deg_ref, hb_ref, wr_ref, br_ref, wt_ref, h_ref):
    h_ref[...] = _layer_math(agg_ref, deg_ref, hb_ref, wr_ref, br_ref, wt_ref)


_agg_spec = pl.BlockSpec((NC, RB, WIDTH), lambda i: (0, i, 0))
_deg_spec = pl.BlockSpec((NC, RB, DEGW), lambda i: (0, i, 0))
_w_spec = _full_spec(WIDTH, WIDTH)
_vec_spec = _full_spec(1, WIDTH)

_tc_layer = pl.pallas_call(
    _tc_layer_body,
    grid=(NPAD // RB,),
    in_specs=[_agg_spec, _deg_spec, _h_spec, _w_spec, _vec_spec, _w_spec,
              _vec_spec, _vec_spec, _vec_spec, _vec_spec],
    out_specs=[_h_spec, _h_spec],
    out_shape=[jax.ShapeDtypeStruct((NPAD, WIDTH), jnp.float32),
               jax.ShapeDtypeStruct((NPAD, WIDTH), jnp.float32)],
)

_tc_last = pl.pallas_call(
    _tc_last_body,
    grid=(NPAD // RB,),
    in_specs=[_agg_spec, _deg_spec, _h_spec, _w_spec, _vec_spec, _w_spec],
    out_specs=_h_spec,
    out_shape=jax.ShapeDtypeStruct((NPAD, WIDTH), jnp.float32),
)


def kernel(x, pos, edge_index, edge_attr, vocab_table, pos_table,
           bn_gamma, bn_beta, bn_mean, bn_var, W_rel, b_rel, W_root):
    del edge_attr
    src = edge_index[0].astype(jnp.int32)
    dst = edge_index[1].astype(jnp.int32)
    padn = jnp.zeros((NPAD - N,), jnp.int32)
    x2 = jnp.concatenate([x.astype(jnp.int32), padn]).reshape(NPAD, 1)
    p2 = jnp.concatenate([pos.astype(jnp.int32), padn]).reshape(NPAD, 1)

    deg_tmp = jax.ops.segment_sum(jnp.ones((E,), jnp.float32), dst,
                                  num_segments=NPAD)
    deg_tmp = jnp.broadcast_to(deg_tmp[:, None], (NPAD, DEGW))
    degp = jnp.stack([deg_tmp, jnp.zeros_like(deg_tmp)])

    row = lambda a: a.reshape(1, WIDTH)
    hb = _tc_embed(x2, p2, vocab_table, pos_table,
                   row(bn_gamma[0]), row(bn_beta[0]),
                   row(bn_mean[0]), row(bn_var[0]))

    h = None
    for i in range(NLAYERS):
        agg_tmp = jax.ops.segment_sum(hb[src], dst, num_segments=NPAD)
        aggp = jnp.stack([agg_tmp, jnp.zeros_like(agg_tmp)])
        wr = W_rel[i].T
        wt = W_root[i].T
        br = b_rel[i].reshape(1, WIDTH)
        if i + 1 < NLAYERS:
            h, hb = _tc_layer(aggp, degp, hb, wr, br, wt,
                              row(bn_gamma[i + 1]), row(bn_beta[i + 1]),
                              row(bn_mean[i + 1]), row(bn_var[i + 1]))
        else:
            h = _tc_last(aggp, degp, hb, wr, br, wt)
    return h[:N]
